# baseline (speedup 1.0000x reference)
import functools

import jax
import jax.numpy as jnp
from jax import lax
from jax.experimental import pallas as pl
from jax.experimental.pallas import tpu as pltpu
from jax.experimental.pallas import tpu_sc as plsc


def _gate_pool_kernel(x_ref, ids_ref, pad_ref, gW1_ref, gb1_ref, gW2_ref,
                      gb2_ref, s_ref, praw_ref):
    x = x_ref[0]
    h = jnp.tanh(jnp.dot(x, gW1_ref[...],
                         preferred_element_type=jnp.float32) + gb1_ref[...])
    g_col = jnp.dot(h, gW2_ref[...],
                    preferred_element_type=jnp.float32)
    g_row = g_col.T + gb2_ref[...]
    e_row = jnp.where(pad_ref[0] != 0, 0.0, jnp.exp(g_row))
    n = x.shape[0]
    c = praw_ref.shape[1]
    onehot = lax.broadcasted_iota(jnp.int32, (c, n), 0) == ids_ref[0]
    numer = jnp.where(onehot, e_row, 0.0)
    s_ref[0] = jnp.sum(numer, axis=1, keepdims=True)
    praw_ref[0] = jnp.dot(numer, x,
                          preferred_element_type=jnp.float32)


def _make_seg_count(BN, C):
    NC, NS, L = 2, 16, 16
    NW = NC * NS
    P = BN // NW
    mesh = plsc.VectorSubcoreMesh(core_axis_name="c", subcore_axis_name="s")

    @functools.partial(
        pl.kernel,
        mesh=mesh,
        out_type=jax.ShapeDtypeStruct((NW, C), jnp.float32),
        compiler_params=pltpu.CompilerParams(needs_layout_passes=False,
                                             skip_device_barrier=True),
        scratch_types=[
            pltpu.VMEM((P,), jnp.int32),
            pltpu.VMEM((P,), jnp.int32),
            pltpu.VMEM((L * C,), jnp.float32),
            pltpu.VMEM((C,), jnp.float32),
        ],
    )
    def seg_count(ids_hbm, pad_hbm, out_hbm, ids_v, pad_v, acc_v, s_v):
        wid = lax.axis_index("s") * NC + lax.axis_index("c")
        base = wid * P
        pltpu.sync_copy(ids_hbm.at[pl.ds(base, P)], ids_v)
        pltpu.sync_copy(pad_hbm.at[pl.ds(base, P)], pad_v)
        zeros = jnp.zeros((L,), jnp.float32)
        for r in range(L * C // L):
            acc_v[pl.ds(r * L, L)] = zeros
        lane_off = lax.iota(jnp.int32, L) * C
        one = jnp.ones((L,), jnp.float32)
        for i in range(P // L):
            nv = one - pad_v[pl.ds(i * L, L)].astype(jnp.float32)
            iv = ids_v[pl.ds(i * L, L)] + lane_off
            av = plsc.load_gather(acc_v, [iv])
            plsc.store_scatter(acc_v, [iv], av + nv)
        for j in range(C // L):
            sv = acc_v[pl.ds(j * L, L)]
            for r in range(1, L):
                sv = sv + acc_v[pl.ds(r * C + j * L, L)]
            s_v[pl.ds(j * L, L)] = sv
        pltpu.sync_copy(s_v, out_hbm.at[wid])

    return seg_count, NW


def _proj_kernel(praw_ref, st_ref, embT_ref, ct_ref, pW_ref, pb_ref,
                 lng_ref, lnb_ref, out_ref):
    c, nb = st_ref.shape
    t = embT_ref.shape[1]
    onehot_tb = (lax.broadcasted_iota(jnp.int32, (t, nb), 0) == ct_ref[...]
                 ).astype(jnp.float32)
    ctmT = jnp.dot(embT_ref[...], onehot_tb,
                   preferred_element_type=jnp.float32)
    st = st_ref[...]
    ne = st > 0.0
    scaleT = jnp.where(ne, (1.0 + 0.1 * ctmT) / jnp.where(ne, st, 1.0), 0.0)
    any_col = jnp.sum(st, axis=1, keepdims=True) > 0.0
    pW = pW_ref[...]
    pb = pb_ref[...]
    lng = lng_ref[...]
    lnb = lnb_ref[...]
    for b in range(nb):
        pooled = praw_ref[b] * scaleT[:, b:b + 1]
        proj = jnp.dot(pooled, pW, preferred_element_type=jnp.float32) + pb
        mean = jnp.mean(proj, axis=1, keepdims=True)
        d = proj - mean
        var = jnp.mean(d * d, axis=1, keepdims=True)
        y = d * lax.rsqrt(var + 1e-5) * lng + lnb
        y = jnp.where(y > 0.0, y, jnp.exp(jnp.minimum(y, 0.0)) - 1.0)
        out_ref[b] = jnp.where(any_col, y, 0.0)


def kernel(x, gW1, gb1, gW2, gb2, emb, pW, pb, ln_g, ln_b, cancer_type,
           channel_ids, pad_mask):
    B, N, H = x.shape
    T, C = emb.shape
    ids_i = channel_ids.astype(jnp.int32)
    pad_i = pad_mask.astype(jnp.int32)

    seg_count, NW = _make_seg_count(B * N, C)
    counts = seg_count(ids_i.reshape(B * N), pad_i.reshape(B * N))
    channel_active = counts.reshape(B, NW // B, C).sum(axis=1) > 0.0

    S, praw = pl.pallas_call(
        _gate_pool_kernel,
        grid=(B,),
        in_specs=[
            pl.BlockSpec((1, N, H), lambda b: (b, 0, 0)),
            pl.BlockSpec((1, 1, N), lambda b: (b, 0, 0)),
            pl.BlockSpec((1, 1, N), lambda b: (b, 0, 0)),
            pl.BlockSpec((H, H // 2), lambda b: (0, 0)),
            pl.BlockSpec((1, H // 2), lambda b: (0, 0)),
            pl.BlockSpec((H // 2, 1), lambda b: (0, 0)),
            pl.BlockSpec((1, 1), lambda b: (0, 0)),
        ],
        out_specs=[
            pl.BlockSpec((1, C, 1), lambda b: (b, 0, 0)),
            pl.BlockSpec((1, C, H), lambda b: (b, 0, 0)),
        ],
        out_shape=[
            jax.ShapeDtypeStruct((B, C, 1), jnp.float32),
            jax.ShapeDtypeStruct((B, C, H), jnp.float32),
        ],
    )(x, ids_i.reshape(B, 1, N), pad_i.reshape(B, 1, N), gW1,
      gb1.reshape(1, -1), gW2, gb2.reshape(1, 1))

    tokens = pl.pallas_call(
        _proj_kernel,
        grid=(1,),
        in_specs=[
            pl.BlockSpec((B, C, H), lambda i: (0, 0, 0)),
            pl.BlockSpec((C, B), lambda i: (0, 0)),
            pl.BlockSpec((C, T), lambda i: (0, 0)),
            pl.BlockSpec((1, B), lambda i: (0, 0)),
            pl.BlockSpec((H, H), lambda i: (0, 0)),
            pl.BlockSpec((1, H), lambda i: (0, 0)),
            pl.BlockSpec((1, H), lambda i: (0, 0)),
            pl.BlockSpec((1, H), lambda i: (0, 0)),
        ],
        out_specs=pl.BlockSpec((B, C, H), lambda i: (0, 0, 0)),
        out_shape=jax.ShapeDtypeStruct((B, C, H), jnp.float32),
    )(praw, S[:, :, 0].T, emb.T, cancer_type.astype(jnp.int32).reshape(1, B),
      pW, pb.reshape(1, -1), ln_g.reshape(1, -1), ln_b.reshape(1, -1))

    return tokens, channel_active


# speedup vs baseline: 1.0974x; 1.0974x over previous
import functools

import jax
import jax.numpy as jnp
from jax import lax
from jax.experimental import pallas as pl
from jax.experimental.pallas import tpu as pltpu
from jax.experimental.pallas import tpu_sc as plsc


def _fused_kernel(x_ref, ids_ref, pad_ref, gW1_ref, gb1_ref, gW2_ref,
                  gb2_ref, embT_ref, ct_ref, pW_ref, pb_ref, lng_ref,
                  lnb_ref, out_ref, praw_s, st_s):
    b = pl.program_id(0)
    nb = ids_ref.shape[0]

    @pl.when(b < nb)
    def _batch_step():
        x = x_ref[0]
        h = jnp.tanh(jnp.dot(x, gW1_ref[...],
                             preferred_element_type=jnp.float32)
                     + gb1_ref[...])
        g_col = jnp.dot(h, gW2_ref[...],
                        preferred_element_type=jnp.float32)
        g_row = g_col.T + gb2_ref[...]
        pad_row = pad_ref[pl.ds(b, 1), :]
        e_row = jnp.where(pad_row != 0, 0.0, jnp.exp(g_row))
        n = x.shape[0]
        c = st_s.shape[1]
        ids_row = ids_ref[pl.ds(b, 1), :]
        onehot = lax.broadcasted_iota(jnp.int32, (c, n), 0) == ids_row
        numer = jnp.where(onehot, e_row, 0.0)
        st_s[b] = jnp.sum(numer, axis=1, keepdims=True)
        praw_s[b] = jnp.dot(numer, x,
                            preferred_element_type=jnp.float32)

    @pl.when(b == nb)
    def _proj_step():
        c = st_s.shape[1]
        t = embT_ref.shape[1]
        onehot_tb = (lax.broadcasted_iota(jnp.int32, (t, nb), 0)
                     == ct_ref[...]).astype(jnp.float32)
        ctmT = jnp.dot(embT_ref[...], onehot_tb,
                       preferred_element_type=jnp.float32)
        st = jnp.concatenate([st_s[i] for i in range(nb)], axis=1)
        ne = st > 0.0
        scaleT = jnp.where(ne, (1.0 + 0.1 * ctmT) / jnp.where(ne, st, 1.0),
                           0.0)
        any_col = jnp.sum(st, axis=1, keepdims=True) > 0.0
        pooled = jnp.concatenate(
            [praw_s[i] * scaleT[:, i:i + 1] for i in range(nb)], axis=0
        )
        proj = jnp.dot(pooled, pW_ref[...],
                       preferred_element_type=jnp.float32) + pb_ref[...]
        mean = jnp.mean(proj, axis=1, keepdims=True)
        d = proj - mean
        var = jnp.mean(d * d, axis=1, keepdims=True)
        y = d * lax.rsqrt(var + 1e-5) * lng_ref[...] + lnb_ref[...]
        y = jnp.where(y > 0.0, y, jnp.exp(jnp.minimum(y, 0.0)) - 1.0)
        anyBC = jnp.concatenate([any_col] * nb, axis=0)
        y = jnp.where(anyBC, y, 0.0)
        out_ref[...] = y.reshape(nb, c, y.shape[1])


def _make_seg_count(BN, C):
    NC, NS, L = 2, 16, 16
    NW = NC * NS
    P = BN // NW
    mesh = plsc.VectorSubcoreMesh(core_axis_name="c", subcore_axis_name="s")

    @functools.partial(
        pl.kernel,
        mesh=mesh,
        out_type=jax.ShapeDtypeStruct((NW, C), jnp.float32),
        compiler_params=pltpu.CompilerParams(needs_layout_passes=False,
                                             skip_device_barrier=True),
        scratch_types=[
            pltpu.VMEM((P,), jnp.int32),
            pltpu.VMEM((P,), jnp.int32),
            pltpu.VMEM((L * C,), jnp.float32),
            pltpu.VMEM((C,), jnp.float32),
        ],
    )
    def seg_count(ids_hbm, pad_hbm, out_hbm, ids_v, pad_v, acc_v, s_v):
        wid = lax.axis_index("s") * NC + lax.axis_index("c")
        base = wid * P
        pltpu.sync_copy(ids_hbm.at[pl.ds(base, P)], ids_v)
        pltpu.sync_copy(pad_hbm.at[pl.ds(base, P)], pad_v)
        zeros = jnp.zeros((L,), jnp.float32)
        for r in range(L * C // L):
            acc_v[pl.ds(r * L, L)] = zeros
        lane_off = lax.iota(jnp.int32, L) * C
        one = jnp.ones((L,), jnp.float32)
        for i in range(P // L):
            nv = one - pad_v[pl.ds(i * L, L)].astype(jnp.float32)
            iv = ids_v[pl.ds(i * L, L)] + lane_off
            av = plsc.load_gather(acc_v, [iv])
            plsc.store_scatter(acc_v, [iv], av + nv)
        for j in range(C // L):
            sv = acc_v[pl.ds(j * L, L)]
            for r in range(1, L):
                sv = sv + acc_v[pl.ds(r * C + j * L, L)]
            s_v[pl.ds(j * L, L)] = sv
        pltpu.sync_copy(s_v, out_hbm.at[wid])

    return seg_count, NW


def kernel(x, gW1, gb1, gW2, gb2, emb, pW, pb, ln_g, ln_b, cancer_type,
           channel_ids, pad_mask):
    B, N, H = x.shape
    T, C = emb.shape
    ids_i = channel_ids.astype(jnp.int32)
    pad_i = pad_mask.astype(jnp.int32)

    seg_count, NW = _make_seg_count(B * N, C)
    counts = seg_count(ids_i.reshape(B * N), pad_i.reshape(B * N))
    channel_active = counts.reshape(B, NW // B, C).sum(axis=1) > 0.0

    tokens = pl.pallas_call(
        _fused_kernel,
        grid=(B + 1,),
        in_specs=[
            pl.BlockSpec((1, N, H), lambda b: (jnp.minimum(b, B - 1), 0, 0)),
            pl.BlockSpec((B, N), lambda b: (0, 0)),
            pl.BlockSpec((B, N), lambda b: (0, 0)),
            pl.BlockSpec((H, H // 2), lambda b: (0, 0)),
            pl.BlockSpec((1, H // 2), lambda b: (0, 0)),
            pl.BlockSpec((H // 2, 1), lambda b: (0, 0)),
            pl.BlockSpec((1, 1), lambda b: (0, 0)),
            pl.BlockSpec((C, T), lambda b: (0, 0)),
            pl.BlockSpec((1, B), lambda b: (0, 0)),
            pl.BlockSpec((H, H), lambda b: (0, 0)),
            pl.BlockSpec((1, H), lambda b: (0, 0)),
            pl.BlockSpec((1, H), lambda b: (0, 0)),
            pl.BlockSpec((1, H), lambda b: (0, 0)),
        ],
        out_specs=pl.BlockSpec((B, C, H), lambda b: (0, 0, 0)),
        out_shape=jax.ShapeDtypeStruct((B, C, H), jnp.float32),
        scratch_shapes=[
            pltpu.VMEM((B, C, H), jnp.float32),
            pltpu.VMEM((B, C, 1), jnp.float32),
        ],
    )(x, ids_i, pad_i, gW1, gb1.reshape(1, -1), gW2, gb2.reshape(1, 1),
      emb.T, cancer_type.astype(jnp.int32).reshape(1, B), pW,
      pb.reshape(1, -1), ln_g.reshape(1, -1), ln_b.reshape(1, -1))

    return tokens, channel_active


# speedup vs baseline: 1.1133x; 1.0145x over previous
import functools

import jax
import jax.numpy as jnp
from jax import lax
from jax.experimental import pallas as pl
from jax.experimental.pallas import tpu as pltpu
from jax.experimental.pallas import tpu_sc as plsc


def _fused_kernel(x_ref, ids_ref, pad_ref, gW1_ref, gb1_ref, gW2_ref,
                  gb2_ref, embT_ref, ct_ref, pW_ref, pb_ref, lng_ref,
                  lnb_ref, out_ref, praw_s, st_s):
    b = pl.program_id(0)
    nb = ids_ref.shape[0]

    @pl.when(b < nb)
    def _batch_step():
        x = x_ref[0]
        h = jnp.tanh(jnp.dot(x, gW1_ref[...],
                             preferred_element_type=jnp.float32)
                     + gb1_ref[...])
        g_col = jnp.dot(h, gW2_ref[...],
                        preferred_element_type=jnp.float32)
        g_row = g_col.T + gb2_ref[...]
        pad_row = pad_ref[pl.ds(b, 1), :]
        e_row = jnp.where(pad_row, 0.0, jnp.exp(g_row))
        n = x.shape[0]
        c = st_s.shape[1]
        ids_row = ids_ref[pl.ds(b, 1), :]
        onehot = lax.broadcasted_iota(jnp.int32, (c, n), 0) == ids_row
        numer = jnp.where(onehot, e_row, 0.0)
        st_s[b] = jnp.sum(numer, axis=1, keepdims=True)
        praw_s[b] = jnp.dot(numer, x,
                            preferred_element_type=jnp.float32)

    @pl.when(b == nb)
    def _proj_step():
        c = st_s.shape[1]
        t = embT_ref.shape[1]
        onehot_tb = (lax.broadcasted_iota(jnp.int32, (t, nb), 0)
                     == ct_ref[...]).astype(jnp.float32)
        ctmT = jnp.dot(embT_ref[...], onehot_tb,
                       preferred_element_type=jnp.float32)
        st = jnp.concatenate([st_s[i] for i in range(nb)], axis=1)
        ne = st > 0.0
        scaleT = jnp.where(ne, (1.0 + 0.1 * ctmT) / jnp.where(ne, st, 1.0),
                           0.0)
        any_col = jnp.sum(st, axis=1, keepdims=True) > 0.0
        pooled = jnp.concatenate(
            [praw_s[i] * scaleT[:, i:i + 1] for i in range(nb)], axis=0
        )
        proj = jnp.dot(pooled, pW_ref[...],
                       preferred_element_type=jnp.float32) + pb_ref[...]
        mean = jnp.mean(proj, axis=1, keepdims=True)
        d = proj - mean
        var = jnp.mean(d * d, axis=1, keepdims=True)
        y = d * lax.rsqrt(var + 1e-5) * lng_ref[...] + lnb_ref[...]
        y = jnp.where(y > 0.0, y, jnp.exp(jnp.minimum(y, 0.0)) - 1.0)
        anyBC = jnp.concatenate([any_col] * nb, axis=0)
        y = jnp.where(anyBC, y, 0.0)
        out_ref[...] = y.reshape(nb, c, y.shape[1])


def _make_seg_count(BN, C):
    NC, NS, L = 2, 16, 16
    NW = NC * NS
    P = BN // NW
    mesh = plsc.VectorSubcoreMesh(core_axis_name="c", subcore_axis_name="s")

    CP = C + L

    @functools.partial(
        pl.kernel,
        mesh=mesh,
        out_type=jax.ShapeDtypeStruct((NW, C), jnp.float32),
        compiler_params=pltpu.CompilerParams(needs_layout_passes=False,
                                             skip_device_barrier=True),
        scratch_types=[
            pltpu.VMEM((P,), jnp.int32),
            pltpu.VMEM((L * (C + L),), jnp.float32),
            pltpu.VMEM((C,), jnp.float32),
        ],
    )
    def seg_count(ids_hbm, out_hbm, ids_v, acc_v, s_v):
        wid = lax.axis_index("s") * NC + lax.axis_index("c")
        base = wid * P
        pltpu.sync_copy(ids_hbm.at[pl.ds(base, P)], ids_v)
        zeros = jnp.zeros((L,), jnp.float32)
        for r in range(L * CP // L):
            acc_v[pl.ds(r * L, L)] = zeros
        lane_off = lax.iota(jnp.int32, L) * CP
        one = jnp.ones((L,), jnp.float32)
        for i in range(P // L):
            iv = ids_v[pl.ds(i * L, L)] + lane_off
            av = plsc.load_gather(acc_v, [iv])
            plsc.store_scatter(acc_v, [iv], av + one)
        for j in range(C // L):
            sv = acc_v[pl.ds(j * L, L)]
            for r in range(1, L):
                sv = sv + acc_v[pl.ds(r * CP + j * L, L)]
            s_v[pl.ds(j * L, L)] = sv
        pltpu.sync_copy(s_v, out_hbm.at[wid])

    return seg_count, NW


def kernel(x, gW1, gb1, gW2, gb2, emb, pW, pb, ln_g, ln_b, cancer_type,
           channel_ids, pad_mask):
    B, N, H = x.shape
    T, C = emb.shape
    ids_i = channel_ids.astype(jnp.int32)
    ids_m_flat = jnp.where(pad_mask, C, ids_i).reshape(B * N)

    seg_count, NW = _make_seg_count(B * N, C)
    counts = seg_count(ids_m_flat)
    channel_active = counts.reshape(B, NW // B, C).sum(axis=1) > 0.0

    tokens = pl.pallas_call(
        _fused_kernel,
        grid=(B + 1,),
        in_specs=[
            pl.BlockSpec((1, N, H), lambda b: (jnp.minimum(b, B - 1), 0, 0)),
            pl.BlockSpec((B, N), lambda b: (0, 0)),
            pl.BlockSpec((B, N), lambda b: (0, 0)),
            pl.BlockSpec((H, H // 2), lambda b: (0, 0)),
            pl.BlockSpec((1, H // 2), lambda b: (0, 0)),
            pl.BlockSpec((H // 2, 1), lambda b: (0, 0)),
            pl.BlockSpec((1, 1), lambda b: (0, 0)),
            pl.BlockSpec((C, T), lambda b: (0, 0)),
            pl.BlockSpec((1, B), lambda b: (0, 0)),
            pl.BlockSpec((H, H), lambda b: (0, 0)),
            pl.BlockSpec((1, H), lambda b: (0, 0)),
            pl.BlockSpec((1, H), lambda b: (0, 0)),
            pl.BlockSpec((1, H), lambda b: (0, 0)),
        ],
        out_specs=pl.BlockSpec((B, C, H), lambda b: (0, 0, 0)),
        out_shape=jax.ShapeDtypeStruct((B, C, H), jnp.float32),
        scratch_shapes=[
            pltpu.VMEM((B, C, H), jnp.float32),
            pltpu.VMEM((B, C, 1), jnp.float32),
        ],
    )(x, ids_i, pad_mask, gW1, gb1.reshape(1, -1), gW2, gb2.reshape(1, 1),
      emb.T, cancer_type.astype(jnp.int32).reshape(1, B), pW,
      pb.reshape(1, -1), ln_g.reshape(1, -1), ln_b.reshape(1, -1))

    return tokens, channel_active


# speedup vs baseline: 1.1418x; 1.0256x over previous
import functools

import jax
import jax.numpy as jnp
from jax import lax
from jax.experimental import pallas as pl
from jax.experimental.pallas import tpu as pltpu
from jax.experimental.pallas import tpu_sc as plsc


def _fused_kernel(x_ref, ids_ref, gW1_ref, gb1_ref, gW2_ref,
                  gb2_ref, embT_ref, ct_ref, pW_ref, pb_ref, lng_ref,
                  lnb_ref, out_ref, praw_s, st_s):
    b = pl.program_id(0)
    nb = ids_ref.shape[0]

    @pl.when(b < nb)
    def _batch_step():
        x = x_ref[0]
        h = jnp.tanh(jnp.dot(x, gW1_ref[...],
                             preferred_element_type=jnp.float32)
                     + gb1_ref[...])
        g_col = jnp.dot(h, gW2_ref[...],
                        preferred_element_type=jnp.float32)
        g_row = g_col.T + gb2_ref[...]
        e_row = jnp.exp(g_row)
        n = x.shape[0]
        c = st_s.shape[1]
        ids_row = ids_ref[pl.ds(b, 1), :]
        onehot = lax.broadcasted_iota(jnp.int32, (c, n), 0) == ids_row
        numer = jnp.where(onehot, e_row, 0.0)
        st_s[b] = jnp.sum(numer, axis=1, keepdims=True)
        praw_s[b] = jnp.dot(numer, x,
                            preferred_element_type=jnp.float32)

    @pl.when(b == nb)
    def _proj_step():
        c = st_s.shape[1]
        t = embT_ref.shape[1]
        onehot_tb = (lax.broadcasted_iota(jnp.int32, (t, nb), 0)
                     == ct_ref[...]).astype(jnp.float32)
        ctmT = jnp.dot(embT_ref[...], onehot_tb,
                       preferred_element_type=jnp.float32)
        st = jnp.concatenate([st_s[i] for i in range(nb)], axis=1)
        ne = st > 0.0
        scaleT = jnp.where(ne, (1.0 + 0.1 * ctmT) / jnp.where(ne, st, 1.0),
                           0.0)
        any_col = jnp.sum(st, axis=1, keepdims=True) > 0.0
        pooled = jnp.concatenate(
            [praw_s[i] * scaleT[:, i:i + 1] for i in range(nb)], axis=0
        )
        proj = jnp.dot(pooled, pW_ref[...],
                       preferred_element_type=jnp.float32) + pb_ref[...]
        mean = jnp.mean(proj, axis=1, keepdims=True)
        d = proj - mean
        var = jnp.mean(d * d, axis=1, keepdims=True)
        y = d * lax.rsqrt(var + 1e-5) * lng_ref[...] + lnb_ref[...]
        y = jnp.where(y > 0.0, y, jnp.exp(jnp.minimum(y, 0.0)) - 1.0)
        anyBC = jnp.concatenate([any_col] * nb, axis=0)
        y = jnp.where(anyBC, y, 0.0)
        out_ref[...] = y.reshape(nb, c, y.shape[1])


def _make_seg_count(BN, C):
    NC, NS, L = 2, 16, 16
    NW = NC * NS
    P = BN // NW
    mesh = plsc.VectorSubcoreMesh(core_axis_name="c", subcore_axis_name="s")

    CP = C + L
    NB = NW // 2

    @functools.partial(
        pl.kernel,
        mesh=mesh,
        out_type=jax.ShapeDtypeStruct((NB, C), jnp.float32),
        compiler_params=pltpu.CompilerParams(needs_layout_passes=False,
                                             skip_device_barrier=True),
        scratch_types=[
            pltpu.VMEM((P,), jnp.int32),
            pltpu.VMEM((L * (C + L),), jnp.float32),
            pltpu.VMEM((C,), jnp.float32),
            pltpu.VMEM((C,), jnp.float32),
            pltpu.VMEM_SHARED((NS, C), jnp.float32),
        ],
    )
    def seg_count(ids_hbm, out_hbm, ids_v, acc_v, s_v, t_v, sh_v):
        cid = lax.axis_index("c")
        sid = lax.axis_index("s")
        wid = cid * NS + sid
        base = wid * P
        pltpu.sync_copy(ids_hbm.at[pl.ds(base, P)], ids_v)
        zeros = jnp.zeros((L,), jnp.float32)
        for r in range(L * CP // L):
            acc_v[pl.ds(r * L, L)] = zeros
        lane_off = lax.iota(jnp.int32, L) * CP
        one = jnp.ones((L,), jnp.float32)
        for i in range(P // L):
            iv = ids_v[pl.ds(i * L, L)] + lane_off
            av = plsc.load_gather(acc_v, [iv])
            plsc.store_scatter(acc_v, [iv], av + one)
        for j in range(C // L):
            sv = acc_v[pl.ds(j * L, L)]
            for r in range(1, L):
                sv = sv + acc_v[pl.ds(r * CP + j * L, L)]
            s_v[pl.ds(j * L, L)] = sv
        pltpu.sync_copy(s_v, sh_v.at[sid])
        plsc.subcore_barrier()

        @pl.when(sid % 2 == 0)
        def _combine():
            pltpu.sync_copy(sh_v.at[sid + 1], t_v)
            for j in range(C // L):
                s_v[pl.ds(j * L, L)] = (s_v[pl.ds(j * L, L)]
                                        + t_v[pl.ds(j * L, L)])
            pltpu.sync_copy(s_v, out_hbm.at[wid // 2])

    return seg_count, NW


def kernel(x, gW1, gb1, gW2, gb2, emb, pW, pb, ln_g, ln_b, cancer_type,
           channel_ids, pad_mask):
    B, N, H = x.shape
    T, C = emb.shape
    ids_m = jnp.where(pad_mask, C, channel_ids.astype(jnp.int32))

    seg_count, NW = _make_seg_count(B * N, C)
    counts = seg_count(ids_m.reshape(B * N))
    channel_active = counts > 0.0

    tokens = pl.pallas_call(
        _fused_kernel,
        grid=(B + 1,),
        in_specs=[
            pl.BlockSpec((1, N, H), lambda b: (jnp.minimum(b, B - 1), 0, 0)),
            pl.BlockSpec((B, N), lambda b: (0, 0)),
            pl.BlockSpec((H, H // 2), lambda b: (0, 0)),
            pl.BlockSpec((1, H // 2), lambda b: (0, 0)),
            pl.BlockSpec((H // 2, 1), lambda b: (0, 0)),
            pl.BlockSpec((1, 1), lambda b: (0, 0)),
            pl.BlockSpec((C, T), lambda b: (0, 0)),
            pl.BlockSpec((1, B), lambda b: (0, 0)),
            pl.BlockSpec((H, H), lambda b: (0, 0)),
            pl.BlockSpec((1, H), lambda b: (0, 0)),
            pl.BlockSpec((1, H), lambda b: (0, 0)),
            pl.BlockSpec((1, H), lambda b: (0, 0)),
        ],
        out_specs=pl.BlockSpec((B, C, H), lambda b: (0, 0, 0)),
        out_shape=jax.ShapeDtypeStruct((B, C, H), jnp.float32),
        scratch_shapes=[
            pltpu.VMEM((B, C, H), jnp.float32),
            pltpu.VMEM((B, C, 1), jnp.float32),
        ],
    )(x, ids_m, gW1, gb1.reshape(1, -1), gW2, gb2.reshape(1, 1),
      emb.T, cancer_type.astype(jnp.int32).reshape(1, B), pW,
      pb.reshape(1, -1), ln_g.reshape(1, -1), ln_b.reshape(1, -1))

    return tokens, channel_active


# speedup vs baseline: 1.1731x; 1.0274x over previous
import functools

import jax
import jax.numpy as jnp
from jax import lax
from jax.experimental import pallas as pl
from jax.experimental.pallas import tpu as pltpu
from jax.experimental.pallas import tpu_sc as plsc


def _fused_kernel(x_ref, ids_ref, gW1_ref, gb1_ref, gW2_ref,
                  gb2_ref, embT_ref, ct_ref, pW_ref, pb_ref, lng_ref,
                  lnb_ref, out_ref, praw_s, st_s):
    b = pl.program_id(0)
    nb = ids_ref.shape[0]

    @pl.when(b < nb)
    def _batch_step():
        x = x_ref[0]
        h = jnp.tanh(jnp.dot(x, gW1_ref[...],
                             preferred_element_type=jnp.float32)
                     + gb1_ref[...])
        g_col = jnp.dot(h, gW2_ref[...],
                        preferred_element_type=jnp.float32)
        g_row = g_col.T + gb2_ref[...]
        e_row = jnp.exp(g_row)
        n = x.shape[0]
        c = st_s.shape[1]
        ids_row = ids_ref[pl.ds(b, 1), :]
        onehot = lax.broadcasted_iota(jnp.int32, (c, n), 0) == ids_row
        numer = jnp.where(onehot, e_row, 0.0)
        st_s[b] = jnp.sum(numer, axis=1, keepdims=True)
        praw_s[b] = jnp.dot(numer, x,
                            preferred_element_type=jnp.float32)

    @pl.when(b == nb)
    def _proj_step():
        c = st_s.shape[1]
        t = embT_ref.shape[0]
        onehot_tb = (lax.broadcasted_iota(jnp.int32, (t, nb), 0)
                     == ct_ref[...]).astype(jnp.float32)
        ctmT = lax.dot_general(embT_ref[...], onehot_tb,
                               (((0,), (0,)), ((), ())),
                               preferred_element_type=jnp.float32)
        st = jnp.concatenate([st_s[i] for i in range(nb)], axis=1)
        ne = st > 0.0
        scaleT = jnp.where(ne, (1.0 + 0.1 * ctmT) / jnp.where(ne, st, 1.0),
                           0.0)
        any_col = jnp.sum(st, axis=1, keepdims=True) > 0.0
        pooled = jnp.concatenate(
            [praw_s[i] * scaleT[:, i:i + 1] for i in range(nb)], axis=0
        )
        proj = jnp.dot(pooled, pW_ref[...],
                       preferred_element_type=jnp.float32) + pb_ref[...]
        mean = jnp.mean(proj, axis=1, keepdims=True)
        d = proj - mean
        var = jnp.mean(d * d, axis=1, keepdims=True)
        y = d * lax.rsqrt(var + 1e-5) * lng_ref[...] + lnb_ref[...]
        y = jnp.where(y > 0.0, y, jnp.exp(jnp.minimum(y, 0.0)) - 1.0)
        anyBC = jnp.concatenate([any_col] * nb, axis=0)
        y = jnp.where(anyBC, y, 0.0)
        out_ref[...] = y.reshape(nb, c, y.shape[1])


def _make_seg_count(BN, C):
    NC, NS, L = 2, 16, 16
    NW = NC * NS
    P = BN // NW
    mesh = plsc.VectorSubcoreMesh(core_axis_name="c", subcore_axis_name="s")

    CP = C + L
    NB = NW // 2

    @functools.partial(
        pl.kernel,
        mesh=mesh,
        out_type=jax.ShapeDtypeStruct((NB, C), jnp.float32),
        compiler_params=pltpu.CompilerParams(needs_layout_passes=False,
                                             skip_device_barrier=True),
        scratch_types=[
            pltpu.VMEM((P,), jnp.int32),
            pltpu.VMEM((L * (C + L),), jnp.float32),
            pltpu.VMEM((C,), jnp.float32),
            pltpu.VMEM((C,), jnp.float32),
            pltpu.VMEM_SHARED((NS, C), jnp.float32),
        ],
    )
    def seg_count(ids_hbm, out_hbm, ids_v, acc_v, s_v, t_v, sh_v):
        cid = lax.axis_index("c")
        sid = lax.axis_index("s")
        wid = cid * NS + sid
        base = wid * P
        pltpu.sync_copy(ids_hbm.at[pl.ds(base, P)], ids_v)
        zeros = jnp.zeros((L,), jnp.float32)
        for r in range(L * CP // L):
            acc_v[pl.ds(r * L, L)] = zeros
        lane_off = lax.iota(jnp.int32, L) * CP
        one = jnp.ones((L,), jnp.float32)
        for i in range(P // L):
            iv = ids_v[pl.ds(i * L, L)] + lane_off
            av = plsc.load_gather(acc_v, [iv])
            plsc.store_scatter(acc_v, [iv], av + one)
        for j in range(C // L):
            sv = acc_v[pl.ds(j * L, L)]
            for r in range(1, L):
                sv = sv + acc_v[pl.ds(r * CP + j * L, L)]
            s_v[pl.ds(j * L, L)] = sv
        pltpu.sync_copy(s_v, sh_v.at[sid])
        plsc.subcore_barrier()

        @pl.when(sid % 2 == 0)
        def _combine():
            pltpu.sync_copy(sh_v.at[sid + 1], t_v)
            for j in range(C // L):
                s_v[pl.ds(j * L, L)] = (s_v[pl.ds(j * L, L)]
                                        + t_v[pl.ds(j * L, L)])
            pltpu.sync_copy(s_v, out_hbm.at[wid // 2])

    return seg_count, NW


def kernel(x, gW1, gb1, gW2, gb2, emb, pW, pb, ln_g, ln_b, cancer_type,
           channel_ids, pad_mask):
    B, N, H = x.shape
    T, C = emb.shape
    ids_m = jnp.where(pad_mask, C, channel_ids.astype(jnp.int32))

    seg_count, NW = _make_seg_count(B * N, C)
    counts = seg_count(ids_m.reshape(B * N))
    channel_active = counts > 0.0

    tokens = pl.pallas_call(
        _fused_kernel,
        grid=(B + 1,),
        in_specs=[
            pl.BlockSpec((1, N, H), lambda b: (jnp.minimum(b, B - 1), 0, 0)),
            pl.BlockSpec((B, N), lambda b: (0, 0)),
            pl.BlockSpec((H, H // 2), lambda b: (0, 0)),
            pl.BlockSpec((1, H // 2), lambda b: (0, 0)),
            pl.BlockSpec((H // 2, 1), lambda b: (0, 0)),
            pl.BlockSpec((1, 1), lambda b: (0, 0)),
            pl.BlockSpec((T, C), lambda b: (0, 0)),
            pl.BlockSpec((1, B), lambda b: (0, 0)),
            pl.BlockSpec((H, H), lambda b: (0, 0)),
            pl.BlockSpec((1, H), lambda b: (0, 0)),
            pl.BlockSpec((1, H), lambda b: (0, 0)),
            pl.BlockSpec((1, H), lambda b: (0, 0)),
        ],
        out_specs=pl.BlockSpec((B, C, H), lambda b: (0, 0, 0)),
        out_shape=jax.ShapeDtypeStruct((B, C, H), jnp.float32),
        scratch_shapes=[
            pltpu.VMEM((B, C, H), jnp.float32),
            pltpu.VMEM((B, C, 1), jnp.float32),
        ],
    )(x, ids_m, gW1, gb1.reshape(1, -1), gW2, gb2.reshape(1, 1),
      emb, cancer_type.astype(jnp.int32).reshape(1, B), pW,
      pb.reshape(1, -1), ln_g.reshape(1, -1), ln_b.reshape(1, -1))

    return tokens, channel_active
